# baseline (speedup 1.0000x reference)
import functools

import jax
import jax.numpy as jnp
from jax import lax
from jax.experimental import pallas as pl
from jax.experimental.pallas import tpu as pltpu
from jax.experimental.pallas import tpu_sc as plsc

N = 10000
E = 320000

NC, NS, L = 2, 16, 16
NW = NC * NS
ET = E + N
C = 128
NCH1 = 165
NCH2 = 84
PT1 = NCH1 * C
PT2 = NCH2 * C
TPAD = NW * PT2 + 4 * C
ZR = 624
ZTAIL = N - NS * ZR

_f32 = jnp.float32


def _zero16():
    return jnp.zeros((L,), _f32)


def _sc_params():
    return pltpu.CompilerParams(needs_layout_passes=False,
                                use_tc_tiling_on_sc=False)


def _sc_l1(a1f, h1h, ep):
    mesh = plsc.VectorSubcoreMesh(core_axis_name="c", subcore_axis_name="s")

    @functools.partial(
        pl.kernel,
        out_type=(
            jax.ShapeDtypeStruct((4 * N, 32), _f32),
            jax.ShapeDtypeStruct((NC * N, 4), _f32),
        ),
        mesh=mesh,
        compiler_params=_sc_params(),
        scratch_types=[
            pltpu.VMEM((8 * N,), _f32),
            pltpu.VMEM((3, 2 * C), jnp.int32),
            pltpu.VMEM((3, C), jnp.int32),
            pltpu.VMEM((3, C), jnp.int32),
            pltpu.VMEM((3, C, 32), _f32),
            pltpu.VMEM((3, C, 4), _f32),
            pltpu.VMEM_SHARED((N, 32), _f32),
            pltpu.VMEM_SHARED((N, 4), _f32),
            pltpu.SemaphoreType.DMA,
            pltpu.SemaphoreType.DMA,
            pltpu.SemaphoreType.DMA,
            pltpu.SemaphoreType.DMA,
            pltpu.SemaphoreType.DMA,
            pltpu.SemaphoreType.DMA,
            pltpu.SemaphoreType.DMA,
            pltpu.SemaphoreType.DMA,
            pltpu.SemaphoreType.DMA,
        ],
    )
    def k(a1f_h, h1_h, ep_h, outp, denp, tab_v, eb_v, dst_v, sg_v, rows_v,
          ws_v, out_sp, den_sp, sem_i0, sem_i1, sem_i2, sem_g0, sem_g1,
          sem_g2, sem_s0, sem_s1, sem_s2):
        core = lax.axis_index("c")
        sub = lax.axis_index("s")
        cgb = sub * NCH1
        coreN = core * N
        iota = lax.iota(jnp.int32, L)
        zi = iota * 0
        r0 = sub * ZR
        sem_i = (sem_i0, sem_i1, sem_i2)
        sem_g = (sem_g0, sem_g1, sem_g2)
        sem_s = (sem_s0, sem_s1, sem_s2)

        pltpu.sync_copy(a1f_h, tab_v)

        def zero_ws():
            for b in range(3):
                for g2 in range(C // L):
                    for hh in range(4):
                        plsc.store_scatter(
                            ws_v.at[b],
                            [g2 * L + iota, jnp.full((L,), hh, jnp.int32)],
                            _zero16())

        zero_ws()
        for kq in range(4):
            pltpu.sync_copy(ws_v.at[0], den_sp.at[pl.ds(r0 + kq * C, C)])
        pltpu.sync_copy(ws_v.at[0, pl.ds(0, ZR - 4 * C)],
                        den_sp.at[pl.ds(r0 + 4 * C, ZR - 4 * C)])

        @pl.when(sub == NS - 1)
        def _():
            pltpu.sync_copy(ws_v.at[0, pl.ds(0, ZTAIL)],
                            den_sp.at[pl.ds(NS * ZR, ZTAIL)])

        for p in range(2):
            hN = (2 * core + p) * N

            if p == 1:
                zero_ws()

            @pl.loop(0, C)
            def _(i):
                for b in range(3):
                    rows_v[b, i, pl.ds(0, L)] = _zero16()
                    rows_v[b, i, pl.ds(L, L)] = _zero16()

            @pl.loop(0, C // L)
            def _(i):
                for b in range(3):
                    dst_v[b, pl.ds(i * L, L)] = zi

            for kq in range(4):
                pltpu.sync_copy(rows_v.at[0],
                                out_sp.at[pl.ds(r0 + kq * C, C)])
            pltpu.sync_copy(rows_v.at[0, pl.ds(0, ZR - 4 * C)],
                            out_sp.at[pl.ds(r0 + 4 * C, ZR - 4 * C)])

            @pl.when(sub == NS - 1)
            def _():
                pltpu.sync_copy(rows_v.at[0, pl.ds(0, ZTAIL)],
                                out_sp.at[pl.ds(NS * ZR, ZTAIL)])

            plsc.subcore_barrier()

            for b in range(3):
                pltpu.async_copy(rows_v.at[b], out_sp.at[dst_v.at[b]],
                                 sem_s[b], add=True)
                pltpu.async_copy(ws_v.at[b], den_sp.at[dst_v.at[b]],
                                 sem_s[b], add=True)

            pltpu.async_copy(ep_h.at[pl.ds(cgb * 2 * C, 2 * C)],
                             eb_v.at[0], sem_i0).wait()
            for g2 in range(C // L):
                sg_v[0, pl.ds(g2 * L, L)] = (
                    eb_v[0, pl.ds(g2 * L, L)] + hN)
            pltpu.async_copy(h1_h.at[sg_v.at[0]], rows_v.at[0], sem_g0)
            pltpu.async_copy(ep_h.at[pl.ds((cgb + 1) * 2 * C, 2 * C)],
                             eb_v.at[1], sem_i1)
            pltpu.async_copy(ep_h.at[pl.ds((cgb + 2) * 2 * C, 2 * C)],
                             eb_v.at[2], sem_i2)

            @pl.loop(0, NCH1, step=3)
            def _(g):
                for b in range(3):
                    nb = (b + 1) % 3
                    cg = cgb + g + b
                    off = cg * C
                    pltpu.make_async_copy(ep_h.at[pl.ds(0, 2 * C)],
                                          eb_v.at[nb], sem_i[nb]).wait()
                    pltpu.make_async_copy(rows_v.at[nb],
                                          out_sp.at[dst_v.at[nb]],
                                          sem_s[nb]).wait()
                    pltpu.make_async_copy(ws_v.at[nb],
                                          den_sp.at[dst_v.at[nb]],
                                          sem_s[nb]).wait()
                    for g2 in range(C // L):
                        sg_v[nb, pl.ds(g2 * L, L)] = (
                            eb_v[nb, pl.ds(g2 * L, L)] + hN)
                    pltpu.async_copy(h1_h.at[sg_v.at[nb]], rows_v.at[nb],
                                     sem_g[nb])
                    for g2 in range(C // L):
                        s16 = eb_v[b, pl.ds(g2 * L, L)]
                        d16 = eb_v[b, pl.ds(C + g2 * L, L)]
                        dst_v[b, pl.ds(g2 * L, L)] = d16
                        valid = (off + g2 * L + iota) < ET
                        av = plsc.load_gather(tab_v,
                                              [(s16 + coreN) * 4 + p])
                        bv = plsc.load_gather(tab_v,
                                              [(d16 + coreN) * 4 + (2 + p)])
                        e = av + bv
                        e = jnp.maximum(e, 0.2 * e)
                        w = jnp.where(valid, jnp.exp(e), 0.0)
                        plsc.store_scatter(
                            ws_v.at[b],
                            [g2 * L + iota, jnp.full((L,), p, jnp.int32)],
                            w)
                    pltpu.async_copy(
                        ep_h.at[pl.ds((cg + 3) * 2 * C, 2 * C)],
                        eb_v.at[b], sem_i[b])
                    pltpu.make_async_copy(h1_h.at[pl.ds(0, C)],
                                          rows_v.at[b], sem_g[b]).wait()

                    @pl.loop(0, C, unroll=8)
                    def _(ei):
                        wspl = plsc.load_gather(ws_v.at[b],
                                                [zi + ei, zi + p])
                        for half in range(2):
                            seg = rows_v[b, ei, pl.ds(half * L, L)]
                            rows_v[b, ei, pl.ds(half * L, L)] = seg * wspl

                    pltpu.async_copy(rows_v.at[b], out_sp.at[dst_v.at[b]],
                                     sem_s[b], add=True)
                    pltpu.async_copy(ws_v.at[b], den_sp.at[dst_v.at[b]],
                                     sem_s[b], add=True)

            for b in range(3):
                pltpu.make_async_copy(rows_v.at[b], out_sp.at[dst_v.at[b]],
                                      sem_s[b]).wait()
                pltpu.make_async_copy(ws_v.at[b], den_sp.at[dst_v.at[b]],
                                      sem_s[b]).wait()
            pltpu.make_async_copy(h1_h.at[pl.ds(0, C)],
                                  rows_v.at[NCH1 % 3],
                                  sem_g[NCH1 % 3]).wait()
            pltpu.make_async_copy(ep_h.at[pl.ds(0, 2 * C)], eb_v.at[1],
                                  sem_i1).wait()
            pltpu.make_async_copy(ep_h.at[pl.ds(0, 2 * C)], eb_v.at[2],
                                  sem_i2).wait()

            plsc.subcore_barrier()
            pltpu.sync_copy(out_sp.at[pl.ds(r0, ZR)],
                            outp.at[pl.ds(hN + r0, ZR)])

            @pl.when(sub == NS - 1)
            def _():
                pltpu.sync_copy(out_sp.at[pl.ds(NS * ZR, ZTAIL)],
                                outp.at[pl.ds(hN + NS * ZR, ZTAIL)])

        pltpu.sync_copy(den_sp.at[pl.ds(r0, ZR)],
                        denp.at[pl.ds(coreN + r0, ZR)])

        @pl.when(sub == NS - 1)
        def _():
            pltpu.sync_copy(den_sp.at[pl.ds(NS * ZR, ZTAIL)],
                            denp.at[pl.ds(coreN + NS * ZR, ZTAIL)])

    return k(a1f, h1h, ep)


def _sc_l2(a2f, h2, ep):
    mesh = plsc.VectorSubcoreMesh(core_axis_name="c", subcore_axis_name="s")

    @functools.partial(
        pl.kernel,
        out_type=jax.ShapeDtypeStruct((NC * N, 20), _f32),
        mesh=mesh,
        compiler_params=_sc_params(),
        scratch_types=[
            pltpu.VMEM((8 * N,), _f32),
            pltpu.VMEM((3, 2 * C), jnp.int32),
            pltpu.VMEM((3, C), jnp.int32),
            pltpu.VMEM((3, C, 16), _f32),
            pltpu.VMEM((3, C, 20), _f32),
            pltpu.VMEM_SHARED((N, 20), _f32),
            pltpu.SemaphoreType.DMA,
            pltpu.SemaphoreType.DMA,
            pltpu.SemaphoreType.DMA,
            pltpu.SemaphoreType.DMA,
            pltpu.SemaphoreType.DMA,
            pltpu.SemaphoreType.DMA,
            pltpu.SemaphoreType.DMA,
            pltpu.SemaphoreType.DMA,
            pltpu.SemaphoreType.DMA,
        ],
    )
    def k(a2f_h, h2_h, ep_h, outp, tab_v, eb_v, dst_v, rows_v, msg_v,
          out_sp, sem_i0, sem_i1, sem_i2, sem_g0, sem_g1, sem_g2, sem_s0,
          sem_s1, sem_s2):
        core = lax.axis_index("c")
        sub = lax.axis_index("s")
        tid = core * NS + sub
        cgb = tid * NCH2
        iota = lax.iota(jnp.int32, L)
        zi = iota * 0
        r0 = sub * ZR
        sem_i = (sem_i0, sem_i1, sem_i2)
        sem_g = (sem_g0, sem_g1, sem_g2)
        sem_s = (sem_s0, sem_s1, sem_s2)

        pltpu.sync_copy(a2f_h, tab_v)

        @pl.loop(0, C)
        def _(i):
            for b in range(3):
                msg_v[b, i, pl.ds(0, L)] = _zero16()
                msg_v[b, i, pl.ds(4, L)] = _zero16()

        @pl.loop(0, C // L)
        def _(i):
            for b in range(3):
                dst_v[b, pl.ds(i * L, L)] = zi

        for kq in range(4):
            pltpu.sync_copy(msg_v.at[0], out_sp.at[pl.ds(r0 + kq * C, C)])
        pltpu.sync_copy(msg_v.at[0, pl.ds(0, ZR - 4 * C)],
                        out_sp.at[pl.ds(r0 + 4 * C, ZR - 4 * C)])

        @pl.when(sub == NS - 1)
        def _():
            pltpu.sync_copy(msg_v.at[0, pl.ds(0, ZTAIL)],
                            out_sp.at[pl.ds(NS * ZR, ZTAIL)])

        plsc.subcore_barrier()

        for b in range(3):
            pltpu.async_copy(msg_v.at[b], out_sp.at[dst_v.at[b]],
                             sem_s[b], add=True)

        pltpu.async_copy(ep_h.at[pl.ds(cgb * 2 * C, 2 * C)],
                         eb_v.at[0], sem_i0).wait()
        pltpu.async_copy(h2_h.at[eb_v.at[0, pl.ds(0, C)]], rows_v.at[0],
                         sem_g0)
        pltpu.async_copy(ep_h.at[pl.ds((cgb + 1) * 2 * C, 2 * C)],
                         eb_v.at[1], sem_i1)
        pltpu.async_copy(ep_h.at[pl.ds((cgb + 2) * 2 * C, 2 * C)],
                         eb_v.at[2], sem_i2)

        @pl.loop(0, NCH2, step=3)
        def _(g):
            for b in range(3):
                nb = (b + 1) % 3
                cg = cgb + g + b
                off = cg * C
                pltpu.make_async_copy(ep_h.at[pl.ds(0, 2 * C)],
                                      eb_v.at[nb], sem_i[nb]).wait()
                pltpu.make_async_copy(msg_v.at[nb],
                                      out_sp.at[dst_v.at[nb]],
                                      sem_s[nb]).wait()
                pltpu.async_copy(h2_h.at[eb_v.at[nb, pl.ds(0, C)]],
                                 rows_v.at[nb], sem_g[nb])
                for g2 in range(C // L):
                    s16 = eb_v[b, pl.ds(g2 * L, L)]
                    d16 = eb_v[b, pl.ds(C + g2 * L, L)]
                    dst_v[b, pl.ds(g2 * L, L)] = d16
                    valid = (off + g2 * L + iota) < ET
                    av = plsc.load_gather(tab_v, [s16 * 8])
                    bv = plsc.load_gather(tab_v, [d16 * 8 + 1])
                    e = av + bv
                    e = jnp.maximum(e, 0.2 * e)
                    w = jnp.where(valid, jnp.exp(e), 0.0)
                    plsc.store_scatter(
                        msg_v.at[b],
                        [g2 * L + iota, jnp.full((L,), 16, jnp.int32)], w)
                pltpu.async_copy(ep_h.at[pl.ds((cg + 3) * 2 * C, 2 * C)],
                                 eb_v.at[b], sem_i[b])
                pltpu.make_async_copy(h2_h.at[pl.ds(0, C)], rows_v.at[b],
                                      sem_g[b]).wait()

                @pl.loop(0, C, unroll=8)
                def _(ei):
                    wspl = plsc.load_gather(msg_v.at[b], [zi + ei, zi + 16])
                    seg = rows_v[b, ei, pl.ds(0, L)]
                    msg_v[b, ei, pl.ds(0, L)] = seg * wspl

                pltpu.async_copy(msg_v.at[b], out_sp.at[dst_v.at[b]],
                                 sem_s[b], add=True)

        for b in range(3):
            pltpu.make_async_copy(msg_v.at[b], out_sp.at[dst_v.at[b]],
                                  sem_s[b]).wait()
        pltpu.make_async_copy(h2_h.at[pl.ds(0, C)], rows_v.at[NCH2 % 3],
                              sem_g[NCH2 % 3]).wait()
        pltpu.make_async_copy(ep_h.at[pl.ds(0, 2 * C)], eb_v.at[1],
                              sem_i1).wait()
        pltpu.make_async_copy(ep_h.at[pl.ds(0, 2 * C)], eb_v.at[2],
                              sem_i2).wait()

        plsc.subcore_barrier()
        pltpu.sync_copy(out_sp.at[pl.ds(r0, ZR)],
                        outp.at[pl.ds(core * N + r0, ZR)])

        @pl.when(sub == NS - 1)
        def _():
            pltpu.sync_copy(out_sp.at[pl.ds(NS * ZR, ZTAIL)],
                            outp.at[pl.ds(core * N + NS * ZR, ZTAIL)])

    return k(a2f, h2, ep)


_BM = 2000


def _tca_body(x_ref, w1_ref, wsk_ref, ac_ref, h1_o, id_o, a_o):
    xb = x_ref[...]
    h1 = jnp.dot(xb, w1_ref[...], preferred_element_type=_f32)
    for h in range(4):
        h1_o[h] = h1[:, h * 32:(h + 1) * 32]
    id_o[...] = jnp.dot(xb, wsk_ref[...], preferred_element_type=_f32)
    a = jnp.dot(h1, ac_ref[...], preferred_element_type=_f32)
    a_o[0] = a[:, :4]
    a_o[1] = a[:, 4:]


def _tc_a(x, W1, Wskip, Acat):
    grid = (N // _BM,)
    return pl.pallas_call(
        _tca_body,
        grid=grid,
        in_specs=[
            pl.BlockSpec((_BM, 128), lambda i: (i, 0)),
            pl.BlockSpec((128, 128), lambda i: (0, 0)),
            pl.BlockSpec((128, 128), lambda i: (0, 0)),
            pl.BlockSpec((128, 8), lambda i: (0, 0)),
        ],
        out_specs=[
            pl.BlockSpec((4, _BM, 32), lambda i: (0, i, 0)),
            pl.BlockSpec((_BM, 128), lambda i: (i, 0)),
            pl.BlockSpec((2, _BM, 4), lambda i: (0, i, 0)),
        ],
        out_shape=[
            jax.ShapeDtypeStruct((4, N, 32), _f32),
            jax.ShapeDtypeStruct((N, 128), _f32),
            jax.ShapeDtypeStruct((2, N, 4), _f32),
        ],
    )(x, W1, Wskip, Acat)


def _tcb_body(op_ref, dp_ref, id_ref, b1_ref, mu_ref, sc_ref, be_ref,
              bsk_ref, r4_ref, w2h_ref, w2a_ref, h2_o, a2_o):
    s = jnp.concatenate([op_ref[0], op_ref[1], op_ref[2], op_ref[3]],
                        axis=-1)
    d4 = jnp.concatenate([dp_ref[0][:, 0:2], dp_ref[1][:, 0:2]], axis=-1)
    dex = jnp.dot(d4, r4_ref[...], preferred_element_type=_f32)
    y = s / (dex + 1e-16) + b1_ref[...]
    y = (y - mu_ref[...]) * sc_ref[...] + be_ref[...]
    y = jnp.where(y > 0, y, jnp.exp(y) - 1.0)
    act = y + id_ref[...] + bsk_ref[...]
    h2_o[...] = jnp.dot(act, w2h_ref[...], preferred_element_type=_f32)
    a2_o[...] = jnp.dot(act, w2a_ref[...], preferred_element_type=_f32)


def _tc_b(outp1, denp1, ident, b1, mu, scl, be, bsk, R4, W2h, W2a):
    grid = (N // _BM,)
    vec = lambda i: (0, 0)
    return pl.pallas_call(
        _tcb_body,
        grid=grid,
        in_specs=[
            pl.BlockSpec((4, _BM, 32), lambda i: (0, i, 0)),
            pl.BlockSpec((2, _BM, 4), lambda i: (0, i, 0)),
            pl.BlockSpec((_BM, 128), lambda i: (i, 0)),
            pl.BlockSpec((1, 128), vec),
            pl.BlockSpec((1, 128), vec),
            pl.BlockSpec((1, 128), vec),
            pl.BlockSpec((1, 128), vec),
            pl.BlockSpec((1, 128), vec),
            pl.BlockSpec((4, 128), vec),
            pl.BlockSpec((128, 16), vec),
            pl.BlockSpec((128, 8), vec),
        ],
        out_specs=[
            pl.BlockSpec((_BM, 16), lambda i: (i, 0)),
            pl.BlockSpec((_BM, 8), lambda i: (i, 0)),
        ],
        out_shape=[
            jax.ShapeDtypeStruct((N, 16), _f32),
            jax.ShapeDtypeStruct((N, 8), _f32),
        ],
    )(outp1, denp1, ident, b1, mu, scl, be, bsk, R4, W2h, W2a)


def _tcc_body(o2_ref, b2_ref, out_o):
    s = o2_ref[0] + o2_ref[1]
    res = s[:, :16] / (s[:, 16:17] + 1e-16) + b2_ref[...]
    m = jnp.max(res, axis=-1, keepdims=True)
    lse = jnp.log(jnp.sum(jnp.exp(res - m), axis=-1, keepdims=True)) + m
    out_o[...] = res - lse


def _tc_c(outp2, b2):
    grid = (N // _BM,)
    return pl.pallas_call(
        _tcc_body,
        grid=grid,
        in_specs=[
            pl.BlockSpec((2, _BM, 20), lambda i: (0, i, 0)),
            pl.BlockSpec((1, 16), lambda i: (0, 0)),
        ],
        out_specs=pl.BlockSpec((_BM, 16), lambda i: (i, 0)),
        out_shape=jax.ShapeDtypeStruct((N, 16), _f32),
    )(outp2, b2)


def kernel(x, W1, att_src1, att_dst1, bias1, bn_gamma, bn_beta, Wskip, bskip,
           W2, att_src2, att_dst2, bias2, bn_mean, bn_var, edge_index):
    loop = jnp.arange(N, dtype=jnp.int32)
    pad = jnp.zeros((TPAD - ET,), jnp.int32)
    srcp = jnp.concatenate([edge_index[0].astype(jnp.int32), loop, pad])
    dstp = jnp.concatenate([edge_index[1].astype(jnp.int32), loop, pad])
    ep = jnp.stack([srcp.reshape(-1, C), dstp.reshape(-1, C)],
                   axis=1).reshape(-1)

    kk = jnp.arange(128)
    hh = kk // 32
    col_as = (hh % 2) + 4 * (hh // 2)
    col_ad = 2 + (hh % 2) + 4 * (hh // 2)
    Acat = jnp.zeros((128, 8), _f32)
    Acat = Acat.at[kk, col_as].set(att_src1.reshape(-1))
    Acat = Acat.at[kk, col_ad].set(att_dst1.reshape(-1))
    R4 = jnp.repeat(jnp.eye(4, dtype=_f32), 32, axis=1)
    W2a = jnp.concatenate(
        [(W2 @ att_src2[0])[:, None], (W2 @ att_dst2[0])[:, None],
         jnp.zeros((128, 6), _f32)], axis=1)

    h1s, ident, acat1 = _tc_a(x, W1, Wskip, Acat)

    outp1, denp1 = _sc_l1(acat1.reshape(-1), h1s.reshape(4 * N, 32), ep)

    bn_scale = bn_gamma * jax.lax.rsqrt(bn_var + 1e-5)
    h2, a2 = _tc_b(outp1.reshape(4, N, 32), denp1.reshape(NC, N, 4),
                   ident, bias1.reshape(1, 128), bn_mean.reshape(1, 128),
                   bn_scale.reshape(1, 128), bn_beta.reshape(1, 128),
                   bskip.reshape(1, 128), R4, W2, W2a)

    outp2 = _sc_l2(a2.reshape(-1), h2, ep)

    return _tc_c(outp2.reshape(NC, N, 20), bias2.reshape(1, 16))


# speedup vs baseline: 1.0118x; 1.0118x over previous
import functools

import jax
import jax.numpy as jnp
from jax import lax
from jax.experimental import pallas as pl
from jax.experimental.pallas import tpu as pltpu
from jax.experimental.pallas import tpu_sc as plsc

N = 10000
E = 320000

NC, NS, L = 2, 16, 16
NW = NC * NS
ET = E + N
C = 128
NCH1 = 165
NCH2 = 84
PT1 = NCH1 * C
PT2 = NCH2 * C
TPAD = NW * PT2 + 4 * C
ZR = 624
ZTAIL = N - NS * ZR

_f32 = jnp.float32


def _zero16():
    return jnp.zeros((L,), _f32)


def _sc_params():
    return pltpu.CompilerParams(needs_layout_passes=False,
                                use_tc_tiling_on_sc=False)


def _sc_l1(a1f, h1h, ep):
    mesh = plsc.VectorSubcoreMesh(core_axis_name="c", subcore_axis_name="s")

    @functools.partial(
        pl.kernel,
        out_type=(
            jax.ShapeDtypeStruct((4 * N, 32), _f32),
            jax.ShapeDtypeStruct((NC * N, 4), _f32),
        ),
        mesh=mesh,
        compiler_params=_sc_params(),
        scratch_types=[
            pltpu.VMEM((8 * N,), _f32),
            pltpu.VMEM((3, 2 * C), jnp.int32),
            pltpu.VMEM((3, C), jnp.int32),
            pltpu.VMEM((3, C), jnp.int32),
            pltpu.VMEM((3, C, 32), _f32),
            pltpu.VMEM((3, C, 4), _f32),
            pltpu.VMEM_SHARED((N, 32), _f32),
            pltpu.VMEM_SHARED((N, 4), _f32),
            pltpu.SemaphoreType.DMA,
            pltpu.SemaphoreType.DMA,
            pltpu.SemaphoreType.DMA,
            pltpu.SemaphoreType.DMA,
            pltpu.SemaphoreType.DMA,
            pltpu.SemaphoreType.DMA,
            pltpu.SemaphoreType.DMA,
            pltpu.SemaphoreType.DMA,
            pltpu.SemaphoreType.DMA,
        ],
    )
    def k(a1f_h, h1_h, ep_h, outp, denp, tab_v, eb_v, dst_v, sg_v, rows_v,
          ws_v, out_sp, den_sp, sem_i0, sem_i1, sem_i2, sem_g0, sem_g1,
          sem_g2, sem_s0, sem_s1, sem_s2):
        core = lax.axis_index("c")
        sub = lax.axis_index("s")
        cgb = sub * NCH1
        coreN = core * N
        iota = lax.iota(jnp.int32, L)
        zi = iota * 0
        r0 = sub * ZR
        sem_i = (sem_i0, sem_i1, sem_i2)
        sem_g = (sem_g0, sem_g1, sem_g2)
        sem_s = (sem_s0, sem_s1, sem_s2)

        pltpu.sync_copy(a1f_h, tab_v)

        def zero_ws():
            for b in range(3):
                for g2 in range(C // L):
                    for hh in range(4):
                        plsc.store_scatter(
                            ws_v.at[b],
                            [g2 * L + iota, jnp.full((L,), hh, jnp.int32)],
                            _zero16())

        zero_ws()
        for kq in range(4):
            pltpu.sync_copy(ws_v.at[0], den_sp.at[pl.ds(r0 + kq * C, C)])
        pltpu.sync_copy(ws_v.at[0, pl.ds(0, ZR - 4 * C)],
                        den_sp.at[pl.ds(r0 + 4 * C, ZR - 4 * C)])

        @pl.when(sub == NS - 1)
        def _():
            pltpu.sync_copy(ws_v.at[0, pl.ds(0, ZTAIL)],
                            den_sp.at[pl.ds(NS * ZR, ZTAIL)])

        for p in range(2):
            hN = (2 * core + p) * N

            if p == 1:
                zero_ws()

            @pl.loop(0, C)
            def _(i):
                for b in range(3):
                    rows_v[b, i, pl.ds(0, L)] = _zero16()
                    rows_v[b, i, pl.ds(L, L)] = _zero16()

            @pl.loop(0, C // L)
            def _(i):
                for b in range(3):
                    dst_v[b, pl.ds(i * L, L)] = zi

            for kq in range(4):
                pltpu.sync_copy(rows_v.at[0],
                                out_sp.at[pl.ds(r0 + kq * C, C)])
            pltpu.sync_copy(rows_v.at[0, pl.ds(0, ZR - 4 * C)],
                            out_sp.at[pl.ds(r0 + 4 * C, ZR - 4 * C)])

            @pl.when(sub == NS - 1)
            def _():
                pltpu.sync_copy(rows_v.at[0, pl.ds(0, ZTAIL)],
                                out_sp.at[pl.ds(NS * ZR, ZTAIL)])

            plsc.subcore_barrier()

            for b in range(3):
                pltpu.async_copy(rows_v.at[b], out_sp.at[dst_v.at[b]],
                                 sem_s[b], add=True)
                pltpu.async_copy(ws_v.at[b], den_sp.at[dst_v.at[b]],
                                 sem_s[b], add=True)

            pltpu.async_copy(ep_h.at[pl.ds(cgb * 2 * C, 2 * C)],
                             eb_v.at[0], sem_i0).wait()
            for g2 in range(C // L):
                sg_v[0, pl.ds(g2 * L, L)] = (
                    eb_v[0, pl.ds(g2 * L, L)] + hN)
            pltpu.async_copy(h1_h.at[sg_v.at[0]], rows_v.at[0], sem_g0)
            pltpu.async_copy(ep_h.at[pl.ds((cgb + 1) * 2 * C, 2 * C)],
                             eb_v.at[1], sem_i1)
            pltpu.async_copy(ep_h.at[pl.ds((cgb + 2) * 2 * C, 2 * C)],
                             eb_v.at[2], sem_i2)

            @pl.loop(0, NCH1, step=3)
            def _(g):
                for b in range(3):
                    nb = (b + 1) % 3
                    cg = cgb + g + b
                    off = cg * C
                    pltpu.make_async_copy(ep_h.at[pl.ds(0, 2 * C)],
                                          eb_v.at[nb], sem_i[nb]).wait()
                    pltpu.make_async_copy(rows_v.at[nb],
                                          out_sp.at[dst_v.at[nb]],
                                          sem_s[nb]).wait()
                    pltpu.make_async_copy(ws_v.at[nb],
                                          den_sp.at[dst_v.at[nb]],
                                          sem_s[nb]).wait()
                    for g2 in range(C // L):
                        sg_v[nb, pl.ds(g2 * L, L)] = (
                            eb_v[nb, pl.ds(g2 * L, L)] + hN)
                    pltpu.async_copy(h1_h.at[sg_v.at[nb]], rows_v.at[nb],
                                     sem_g[nb])
                    for g2 in range(C // L):
                        s16 = eb_v[b, pl.ds(g2 * L, L)]
                        d16 = eb_v[b, pl.ds(C + g2 * L, L)]
                        dst_v[b, pl.ds(g2 * L, L)] = d16
                        valid = (off + g2 * L + iota) < ET
                        av = plsc.load_gather(tab_v,
                                              [(s16 + coreN) * 4 + p])
                        bv = plsc.load_gather(tab_v,
                                              [(d16 + coreN) * 4 + (2 + p)])
                        e = av + bv
                        e = jnp.maximum(e, 0.2 * e)
                        w = jnp.where(valid, jnp.exp(e), 0.0)
                        plsc.store_scatter(
                            ws_v.at[b],
                            [g2 * L + iota, jnp.full((L,), p, jnp.int32)],
                            w)
                    pltpu.async_copy(
                        ep_h.at[pl.ds((cg + 3) * 2 * C, 2 * C)],
                        eb_v.at[b], sem_i[b])
                    pltpu.make_async_copy(h1_h.at[pl.ds(0, C)],
                                          rows_v.at[b], sem_g[b]).wait()

                    @pl.loop(0, C, unroll=16)
                    def _(ei):
                        wspl = plsc.load_gather(ws_v.at[b],
                                                [zi + ei, zi + p])
                        for half in range(2):
                            seg = rows_v[b, ei, pl.ds(half * L, L)]
                            rows_v[b, ei, pl.ds(half * L, L)] = seg * wspl

                    pltpu.async_copy(rows_v.at[b], out_sp.at[dst_v.at[b]],
                                     sem_s[b], add=True)
                    pltpu.async_copy(ws_v.at[b], den_sp.at[dst_v.at[b]],
                                     sem_s[b], add=True)

            for b in range(3):
                pltpu.make_async_copy(rows_v.at[b], out_sp.at[dst_v.at[b]],
                                      sem_s[b]).wait()
                pltpu.make_async_copy(ws_v.at[b], den_sp.at[dst_v.at[b]],
                                      sem_s[b]).wait()
            pltpu.make_async_copy(h1_h.at[pl.ds(0, C)],
                                  rows_v.at[NCH1 % 3],
                                  sem_g[NCH1 % 3]).wait()
            pltpu.make_async_copy(ep_h.at[pl.ds(0, 2 * C)], eb_v.at[1],
                                  sem_i1).wait()
            pltpu.make_async_copy(ep_h.at[pl.ds(0, 2 * C)], eb_v.at[2],
                                  sem_i2).wait()

            plsc.subcore_barrier()
            pltpu.sync_copy(out_sp.at[pl.ds(r0, ZR)],
                            outp.at[pl.ds(hN + r0, ZR)])

            @pl.when(sub == NS - 1)
            def _():
                pltpu.sync_copy(out_sp.at[pl.ds(NS * ZR, ZTAIL)],
                                outp.at[pl.ds(hN + NS * ZR, ZTAIL)])

        pltpu.sync_copy(den_sp.at[pl.ds(r0, ZR)],
                        denp.at[pl.ds(coreN + r0, ZR)])

        @pl.when(sub == NS - 1)
        def _():
            pltpu.sync_copy(den_sp.at[pl.ds(NS * ZR, ZTAIL)],
                            denp.at[pl.ds(coreN + NS * ZR, ZTAIL)])

    return k(a1f, h1h, ep)


def _sc_l2(a2f, h2, ep):
    mesh = plsc.VectorSubcoreMesh(core_axis_name="c", subcore_axis_name="s")

    @functools.partial(
        pl.kernel,
        out_type=jax.ShapeDtypeStruct((NC * N, 20), _f32),
        mesh=mesh,
        compiler_params=_sc_params(),
        scratch_types=[
            pltpu.VMEM((8 * N,), _f32),
            pltpu.VMEM((3, 2 * C), jnp.int32),
            pltpu.VMEM((3, C), jnp.int32),
            pltpu.VMEM((3, C, 16), _f32),
            pltpu.VMEM((3, C, 20), _f32),
            pltpu.VMEM_SHARED((N, 20), _f32),
            pltpu.SemaphoreType.DMA,
            pltpu.SemaphoreType.DMA,
            pltpu.SemaphoreType.DMA,
            pltpu.SemaphoreType.DMA,
            pltpu.SemaphoreType.DMA,
            pltpu.SemaphoreType.DMA,
            pltpu.SemaphoreType.DMA,
            pltpu.SemaphoreType.DMA,
            pltpu.SemaphoreType.DMA,
        ],
    )
    def k(a2f_h, h2_h, ep_h, outp, tab_v, eb_v, dst_v, rows_v, msg_v,
          out_sp, sem_i0, sem_i1, sem_i2, sem_g0, sem_g1, sem_g2, sem_s0,
          sem_s1, sem_s2):
        core = lax.axis_index("c")
        sub = lax.axis_index("s")
        tid = core * NS + sub
        cgb = tid * NCH2
        iota = lax.iota(jnp.int32, L)
        zi = iota * 0
        r0 = sub * ZR
        sem_i = (sem_i0, sem_i1, sem_i2)
        sem_g = (sem_g0, sem_g1, sem_g2)
        sem_s = (sem_s0, sem_s1, sem_s2)

        pltpu.sync_copy(a2f_h, tab_v)

        @pl.loop(0, C)
        def _(i):
            for b in range(3):
                msg_v[b, i, pl.ds(0, L)] = _zero16()
                msg_v[b, i, pl.ds(4, L)] = _zero16()

        @pl.loop(0, C // L)
        def _(i):
            for b in range(3):
                dst_v[b, pl.ds(i * L, L)] = zi

        for kq in range(4):
            pltpu.sync_copy(msg_v.at[0], out_sp.at[pl.ds(r0 + kq * C, C)])
        pltpu.sync_copy(msg_v.at[0, pl.ds(0, ZR - 4 * C)],
                        out_sp.at[pl.ds(r0 + 4 * C, ZR - 4 * C)])

        @pl.when(sub == NS - 1)
        def _():
            pltpu.sync_copy(msg_v.at[0, pl.ds(0, ZTAIL)],
                            out_sp.at[pl.ds(NS * ZR, ZTAIL)])

        plsc.subcore_barrier()

        for b in range(3):
            pltpu.async_copy(msg_v.at[b], out_sp.at[dst_v.at[b]],
                             sem_s[b], add=True)

        pltpu.async_copy(ep_h.at[pl.ds(cgb * 2 * C, 2 * C)],
                         eb_v.at[0], sem_i0).wait()
        pltpu.async_copy(h2_h.at[eb_v.at[0, pl.ds(0, C)]], rows_v.at[0],
                         sem_g0)
        pltpu.async_copy(ep_h.at[pl.ds((cgb + 1) * 2 * C, 2 * C)],
                         eb_v.at[1], sem_i1)
        pltpu.async_copy(ep_h.at[pl.ds((cgb + 2) * 2 * C, 2 * C)],
                         eb_v.at[2], sem_i2)

        @pl.loop(0, NCH2, step=3)
        def _(g):
            for b in range(3):
                nb = (b + 1) % 3
                cg = cgb + g + b
                off = cg * C
                pltpu.make_async_copy(ep_h.at[pl.ds(0, 2 * C)],
                                      eb_v.at[nb], sem_i[nb]).wait()
                pltpu.make_async_copy(msg_v.at[nb],
                                      out_sp.at[dst_v.at[nb]],
                                      sem_s[nb]).wait()
                pltpu.async_copy(h2_h.at[eb_v.at[nb, pl.ds(0, C)]],
                                 rows_v.at[nb], sem_g[nb])
                for g2 in range(C // L):
                    s16 = eb_v[b, pl.ds(g2 * L, L)]
                    d16 = eb_v[b, pl.ds(C + g2 * L, L)]
                    dst_v[b, pl.ds(g2 * L, L)] = d16
                    valid = (off + g2 * L + iota) < ET
                    av = plsc.load_gather(tab_v, [s16 * 8])
                    bv = plsc.load_gather(tab_v, [d16 * 8 + 1])
                    e = av + bv
                    e = jnp.maximum(e, 0.2 * e)
                    w = jnp.where(valid, jnp.exp(e), 0.0)
                    plsc.store_scatter(
                        msg_v.at[b],
                        [g2 * L + iota, jnp.full((L,), 16, jnp.int32)], w)
                pltpu.async_copy(ep_h.at[pl.ds((cg + 3) * 2 * C, 2 * C)],
                                 eb_v.at[b], sem_i[b])
                pltpu.make_async_copy(h2_h.at[pl.ds(0, C)], rows_v.at[b],
                                      sem_g[b]).wait()

                @pl.loop(0, C, unroll=16)
                def _(ei):
                    wspl = plsc.load_gather(msg_v.at[b], [zi + ei, zi + 16])
                    seg = rows_v[b, ei, pl.ds(0, L)]
                    msg_v[b, ei, pl.ds(0, L)] = seg * wspl

                pltpu.async_copy(msg_v.at[b], out_sp.at[dst_v.at[b]],
                                 sem_s[b], add=True)

        for b in range(3):
            pltpu.make_async_copy(msg_v.at[b], out_sp.at[dst_v.at[b]],
                                  sem_s[b]).wait()
        pltpu.make_async_copy(h2_h.at[pl.ds(0, C)], rows_v.at[NCH2 % 3],
                              sem_g[NCH2 % 3]).wait()
        pltpu.make_async_copy(ep_h.at[pl.ds(0, 2 * C)], eb_v.at[1],
                              sem_i1).wait()
        pltpu.make_async_copy(ep_h.at[pl.ds(0, 2 * C)], eb_v.at[2],
                              sem_i2).wait()

        plsc.subcore_barrier()
        pltpu.sync_copy(out_sp.at[pl.ds(r0, ZR)],
                        outp.at[pl.ds(core * N + r0, ZR)])

        @pl.when(sub == NS - 1)
        def _():
            pltpu.sync_copy(out_sp.at[pl.ds(NS * ZR, ZTAIL)],
                            outp.at[pl.ds(core * N + NS * ZR, ZTAIL)])

    return k(a2f, h2, ep)


_BM = 2000


def _tca_body(x_ref, w1_ref, wsk_ref, ac_ref, h1_o, id_o, a_o):
    xb = x_ref[...]
    h1 = jnp.dot(xb, w1_ref[...], preferred_element_type=_f32)
    for h in range(4):
        h1_o[h] = h1[:, h * 32:(h + 1) * 32]
    id_o[...] = jnp.dot(xb, wsk_ref[...], preferred_element_type=_f32)
    a = jnp.dot(h1, ac_ref[...], preferred_element_type=_f32)
    a_o[0] = a[:, :4]
    a_o[1] = a[:, 4:]


def _tc_a(x, W1, Wskip, Acat):
    grid = (N // _BM,)
    return pl.pallas_call(
        _tca_body,
        grid=grid,
        in_specs=[
            pl.BlockSpec((_BM, 128), lambda i: (i, 0)),
            pl.BlockSpec((128, 128), lambda i: (0, 0)),
            pl.BlockSpec((128, 128), lambda i: (0, 0)),
            pl.BlockSpec((128, 8), lambda i: (0, 0)),
        ],
        out_specs=[
            pl.BlockSpec((4, _BM, 32), lambda i: (0, i, 0)),
            pl.BlockSpec((_BM, 128), lambda i: (i, 0)),
            pl.BlockSpec((2, _BM, 4), lambda i: (0, i, 0)),
        ],
        out_shape=[
            jax.ShapeDtypeStruct((4, N, 32), _f32),
            jax.ShapeDtypeStruct((N, 128), _f32),
            jax.ShapeDtypeStruct((2, N, 4), _f32),
        ],
    )(x, W1, Wskip, Acat)


def _tcb_body(op_ref, dp_ref, id_ref, b1_ref, mu_ref, sc_ref, be_ref,
              bsk_ref, r4_ref, w2h_ref, w2a_ref, h2_o, a2_o):
    s = jnp.concatenate([op_ref[0], op_ref[1], op_ref[2], op_ref[3]],
                        axis=-1)
    d4 = jnp.concatenate([dp_ref[0][:, 0:2], dp_ref[1][:, 0:2]], axis=-1)
    dex = jnp.dot(d4, r4_ref[...], preferred_element_type=_f32)
    y = s / (dex + 1e-16) + b1_ref[...]
    y = (y - mu_ref[...]) * sc_ref[...] + be_ref[...]
    y = jnp.where(y > 0, y, jnp.exp(y) - 1.0)
    act = y + id_ref[...] + bsk_ref[...]
    h2_o[...] = jnp.dot(act, w2h_ref[...], preferred_element_type=_f32)
    a2_o[...] = jnp.dot(act, w2a_ref[...], preferred_element_type=_f32)


def _tc_b(outp1, denp1, ident, b1, mu, scl, be, bsk, R4, W2h, W2a):
    grid = (N // _BM,)
    vec = lambda i: (0, 0)
    return pl.pallas_call(
        _tcb_body,
        grid=grid,
        in_specs=[
            pl.BlockSpec((4, _BM, 32), lambda i: (0, i, 0)),
            pl.BlockSpec((2, _BM, 4), lambda i: (0, i, 0)),
            pl.BlockSpec((_BM, 128), lambda i: (i, 0)),
            pl.BlockSpec((1, 128), vec),
            pl.BlockSpec((1, 128), vec),
            pl.BlockSpec((1, 128), vec),
            pl.BlockSpec((1, 128), vec),
            pl.BlockSpec((1, 128), vec),
            pl.BlockSpec((4, 128), vec),
            pl.BlockSpec((128, 16), vec),
            pl.BlockSpec((128, 8), vec),
        ],
        out_specs=[
            pl.BlockSpec((_BM, 16), lambda i: (i, 0)),
            pl.BlockSpec((_BM, 8), lambda i: (i, 0)),
        ],
        out_shape=[
            jax.ShapeDtypeStruct((N, 16), _f32),
            jax.ShapeDtypeStruct((N, 8), _f32),
        ],
    )(outp1, denp1, ident, b1, mu, scl, be, bsk, R4, W2h, W2a)


def _tcc_body(o2_ref, b2_ref, out_o):
    s = o2_ref[0] + o2_ref[1]
    res = s[:, :16] / (s[:, 16:17] + 1e-16) + b2_ref[...]
    m = jnp.max(res, axis=-1, keepdims=True)
    lse = jnp.log(jnp.sum(jnp.exp(res - m), axis=-1, keepdims=True)) + m
    out_o[...] = res - lse


def _tc_c(outp2, b2):
    grid = (N // _BM,)
    return pl.pallas_call(
        _tcc_body,
        grid=grid,
        in_specs=[
            pl.BlockSpec((2, _BM, 20), lambda i: (0, i, 0)),
            pl.BlockSpec((1, 16), lambda i: (0, 0)),
        ],
        out_specs=pl.BlockSpec((_BM, 16), lambda i: (i, 0)),
        out_shape=jax.ShapeDtypeStruct((N, 16), _f32),
    )(outp2, b2)


def kernel(x, W1, att_src1, att_dst1, bias1, bn_gamma, bn_beta, Wskip, bskip,
           W2, att_src2, att_dst2, bias2, bn_mean, bn_var, edge_index):
    loop = jnp.arange(N, dtype=jnp.int32)
    pad = jnp.zeros((TPAD - ET,), jnp.int32)
    srcp = jnp.concatenate([edge_index[0].astype(jnp.int32), loop, pad])
    dstp = jnp.concatenate([edge_index[1].astype(jnp.int32), loop, pad])
    ep = jnp.stack([srcp.reshape(-1, C), dstp.reshape(-1, C)],
                   axis=1).reshape(-1)

    kk = jnp.arange(128)
    hh = kk // 32
    col_as = (hh % 2) + 4 * (hh // 2)
    col_ad = 2 + (hh % 2) + 4 * (hh // 2)
    Acat = jnp.zeros((128, 8), _f32)
    Acat = Acat.at[kk, col_as].set(att_src1.reshape(-1))
    Acat = Acat.at[kk, col_ad].set(att_dst1.reshape(-1))
    R4 = jnp.repeat(jnp.eye(4, dtype=_f32), 32, axis=1)
    W2a = jnp.concatenate(
        [(W2 @ att_src2[0])[:, None], (W2 @ att_dst2[0])[:, None],
         jnp.zeros((128, 6), _f32)], axis=1)

    h1s, ident, acat1 = _tc_a(x, W1, Wskip, Acat)

    outp1, denp1 = _sc_l1(acat1.reshape(-1), h1s.reshape(4 * N, 32), ep)

    bn_scale = bn_gamma * jax.lax.rsqrt(bn_var + 1e-5)
    h2, a2 = _tc_b(outp1.reshape(4, N, 32), denp1.reshape(NC, N, 4),
                   ident, bias1.reshape(1, 128), bn_mean.reshape(1, 128),
                   bn_scale.reshape(1, 128), bn_beta.reshape(1, 128),
                   bskip.reshape(1, 128), R4, W2, W2a)

    outp2 = _sc_l2(a2.reshape(-1), h2, ep)

    return _tc_c(outp2.reshape(NC, N, 20), bias2.reshape(1, 16))
